# Initial kernel scaffold; baseline (speedup 1.0000x reference)
import jax
import jax.numpy as jnp
from jax.experimental import pallas as pl


def kernel(pos, edge_index):
    raise NotImplementedError("write your pallas kernel here")


# speedup vs baseline: 8.0366x; 8.0366x over previous
import functools

import jax
import jax.numpy as jnp
from jax import lax
from jax.experimental import pallas as pl
from jax.experimental.pallas import tpu as pltpu
from jax.experimental.pallas import tpu_sc as plsc

_NC = 2
_NS = 16
_W = _NC * _NS
_L = 16


def _rsqrt(s):
    bits = lax.bitcast_convert_type(s, jnp.int32)
    bits = jnp.int32(0x5F3759DF) - (bits >> 1)
    y = lax.bitcast_convert_type(bits, jnp.float32)
    for _ in range(3):
        y = y * (1.5 - 0.5 * s * y * y)
    return y


def _edge_body(E, C, eidx, px, py, pz, out,
               jv, iv, xj, yj, zj, xi, yi, zi, ov, sem_j, sem_i):
    per_w = E // _W
    n_chunks = per_w // C
    wid = lax.axis_index("s") * _NC + lax.axis_index("c")
    lane = lax.iota(jnp.int32, _L)

    def chunk(t, carry):
        base = wid * per_w + t * C
        pltpu.sync_copy(eidx.at[pl.ds(base, C)], jv)
        pltpu.sync_copy(eidx.at[pl.ds(E + base, C)], iv)
        cps = [
            pltpu.async_copy(px.at[jv], xj, sem_j),
            pltpu.async_copy(py.at[jv], yj, sem_j),
            pltpu.async_copy(pz.at[jv], zj, sem_j),
            pltpu.async_copy(px.at[iv], xi, sem_i),
            pltpu.async_copy(py.at[iv], yi, sem_i),
            pltpu.async_copy(pz.at[iv], zi, sem_i),
        ]
        for cp in cps:
            cp.wait()

        def grp(g, carry2):
            sl = pl.ds(g * _L, _L)
            dx = xj[sl] - xi[sl]
            dy = yj[sl] - yi[sl]
            dz = zj[sl] - zi[sl]
            s = dx * dx + dy * dy + dz * dz
            r = _rsqrt(s)
            flat = (g * _L + lane) * 4
            plsc.store_scatter(ov, [flat], s * r)
            plsc.store_scatter(ov, [flat + 1], dx * r)
            plsc.store_scatter(ov, [flat + 2], dy * r)
            plsc.store_scatter(ov, [flat + 3], dz * r)
            return carry2

        lax.fori_loop(0, C // _L, grp, 0)
        pltpu.sync_copy(ov, out.at[pl.ds(4 * base, 4 * C)])
        return carry

    lax.fori_loop(0, n_chunks, chunk, 0)


def kernel(pos, edge_index):
    N = pos.shape[0]
    E = edge_index.shape[1]
    assert E % _W == 0
    per_w = E // _W
    C = 2000
    assert per_w % C == 0 and C % _L == 0
    post = pos.T

    mesh = plsc.VectorSubcoreMesh(core_axis_name="c", subcore_axis_name="s")
    k = pl.kernel(
        functools.partial(_edge_body, E, C),
        out_type=jax.ShapeDtypeStruct((4 * E,), jnp.float32),
        mesh=mesh,
        compiler_params=pltpu.CompilerParams(needs_layout_passes=False),
        scratch_types=[
            pltpu.VMEM((C,), jnp.int32),
            pltpu.VMEM((C,), jnp.int32),
            pltpu.VMEM((C,), jnp.float32),
            pltpu.VMEM((C,), jnp.float32),
            pltpu.VMEM((C,), jnp.float32),
            pltpu.VMEM((C,), jnp.float32),
            pltpu.VMEM((C,), jnp.float32),
            pltpu.VMEM((C,), jnp.float32),
            pltpu.VMEM((4 * C,), jnp.float32),
            pltpu.SemaphoreType.DMA,
            pltpu.SemaphoreType.DMA,
        ],
    )
    out_flat = k(edge_index.reshape(2 * E), post[0], post[1], post[2])
    return out_flat.reshape(E, 4)


# speedup vs baseline: 9.7857x; 1.2176x over previous
import functools

import jax
import jax.numpy as jnp
from jax import lax
from jax.experimental import pallas as pl
from jax.experimental.pallas import tpu as pltpu
from jax.experimental.pallas import tpu_sc as plsc

_NC = 2
_NS = 16
_W = _NC * _NS
_L = 16


def _rsqrt(s):
    bits = lax.bitcast_convert_type(s, jnp.int32)
    bits = jnp.int32(0x5F3759DF) - (bits >> 1)
    y = lax.bitcast_convert_type(bits, jnp.float32)
    for _ in range(3):
        y = y * (1.5 - 0.5 * s * y * y)
    return y


def _edge_body(E, C, eidx, posp, out,
               jv, iv, pj, pi, ov, sem_idx, sem_g, sem_out):
    per_w = E // _W
    n_chunks = per_w // C
    wid = lax.axis_index("s") * _NC + lax.axis_index("c")
    w0 = wid * per_w
    lane = lax.iota(jnp.int32, _L)

    def idx_copy(t, b):
        base = w0 + t * C
        pltpu.async_copy(eidx.at[pl.ds(base, C)], jv[b], sem_idx[b])
        pltpu.async_copy(eidx.at[pl.ds(E + base, C)], iv[b], sem_idx[b])

    def idx_wait(b):
        pltpu.make_async_copy(eidx.at[pl.ds(0, C)], jv[b], sem_idx[b]).wait()
        pltpu.make_async_copy(eidx.at[pl.ds(0, C)], iv[b], sem_idx[b]).wait()

    def gather(b):
        pltpu.async_copy(posp.at[jv[b]], pj[b], sem_g[b])
        pltpu.async_copy(posp.at[iv[b]], pi[b], sem_g[b])

    def gather_wait(b):
        pltpu.make_async_copy(posp.at[jv[b]], pj[b], sem_g[b]).wait()
        pltpu.make_async_copy(posp.at[iv[b]], pi[b], sem_g[b]).wait()

    def out_copy(t, b):
        pltpu.async_copy(ov[b], out.at[pl.ds(4 * (w0 + t * C), 4 * C)],
                         sem_out[b])

    def out_wait(b):
        pltpu.make_async_copy(ov[b], out.at[pl.ds(0, 4 * C)],
                              sem_out[b]).wait()

    def compute(b):
        def grp(g, carry2):
            rows = g * _L + lane
            r4 = rows * 4
            xj = plsc.load_gather(pj[b], [rows, lane * 0])
            yj = plsc.load_gather(pj[b], [rows, lane * 0 + 1])
            zj = plsc.load_gather(pj[b], [rows, lane * 0 + 2])
            xi = plsc.load_gather(pi[b], [rows, lane * 0])
            yi = plsc.load_gather(pi[b], [rows, lane * 0 + 1])
            zi = plsc.load_gather(pi[b], [rows, lane * 0 + 2])
            dx = xj - xi
            dy = yj - yi
            dz = zj - zi
            s = dx * dx + dy * dy + dz * dz
            r = _rsqrt(s)
            plsc.store_scatter(ov[b], [r4], s * r)
            plsc.store_scatter(ov[b], [r4 + 1], dx * r)
            plsc.store_scatter(ov[b], [r4 + 2], dy * r)
            plsc.store_scatter(ov[b], [r4 + 3], dz * r)
            return carry2

        lax.fori_loop(0, C // _L, grp, 0)

    idx_copy(0, 0)
    idx_copy(1, 1)
    idx_wait(0)
    gather(0)

    def step(t, b):
        nb = 1 - b

        @pl.when(t + 1 < n_chunks)
        def _():
            idx_wait(nb)
            gather(nb)

        gather_wait(b)

        @pl.when(t >= 2)
        def _():
            out_wait(b)

        compute(b)
        out_copy(t, b)

        @pl.when(t + 2 < n_chunks)
        def _():
            idx_copy(t + 2, b)

    def pair(p, carry):
        step(2 * p, 0)
        step(2 * p + 1, 1)
        return carry

    assert n_chunks % 2 == 0
    lax.fori_loop(0, n_chunks // 2, pair, 0)
    out_wait(0)
    out_wait(1)


def kernel(pos, edge_index):
    N = pos.shape[0]
    E = edge_index.shape[1]
    assert E % _W == 0
    per_w = E // _W
    C = 2000
    assert per_w % C == 0 and C % _L == 0
    posp = jnp.concatenate(
        [pos, jnp.zeros((N, 1), jnp.float32)], axis=1)

    mesh = plsc.VectorSubcoreMesh(core_axis_name="c", subcore_axis_name="s")
    k = pl.kernel(
        functools.partial(_edge_body, E, C),
        out_type=jax.ShapeDtypeStruct((4 * E,), jnp.float32),
        mesh=mesh,
        compiler_params=pltpu.CompilerParams(
            needs_layout_passes=False, use_tc_tiling_on_sc=False),
        scratch_types=[
            [pltpu.VMEM((C,), jnp.int32)] * 2,
            [pltpu.VMEM((C,), jnp.int32)] * 2,
            [pltpu.VMEM((C, 4), jnp.float32)] * 2,
            [pltpu.VMEM((C, 4), jnp.float32)] * 2,
            [pltpu.VMEM((4 * C,), jnp.float32)] * 2,
            [pltpu.SemaphoreType.DMA] * 2,
            [pltpu.SemaphoreType.DMA] * 2,
            [pltpu.SemaphoreType.DMA] * 2,
        ],
    )
    out_flat = k(edge_index.reshape(2 * E), posp)
    return out_flat.reshape(E, 4)


# speedup vs baseline: 27.7486x; 2.8356x over previous
import functools

import jax
import jax.numpy as jnp
from jax import lax
from jax.experimental import pallas as pl
from jax.experimental.pallas import tpu as pltpu
from jax.experimental.pallas import tpu_sc as plsc

_NC = 2
_NS = 16
_W = _NC * _NS
_L = 16
_B = 128


def _rsqrt(s):
    bits = lax.bitcast_convert_type(s, jnp.int32)
    bits = jnp.int32(0x5F3759DF) - (bits >> 1)
    y = lax.bitcast_convert_type(bits, jnp.float32)
    for _ in range(3):
        y = y * (1.5 - 0.5 * s * y * y)
    return y


def _edge_body(E, C, eidx, px, py, pz, out,
               jv, iv, xj, yj, zj, xi, yi, zi, ov, sem_j, sem_i):
    n_chunks = E // C
    n_steps = (n_chunks + _W - 1) // _W
    wid = lax.axis_index("s") * _NC + lax.axis_index("c")
    lane = lax.iota(jnp.int32, _L)

    def chunk(t, carry):
        gc = wid + t * _W

        @pl.when(gc < n_chunks)
        def _():
            base = gc * C
            pltpu.sync_copy(eidx.at[pl.ds(base, C)], jv)
            pltpu.sync_copy(eidx.at[pl.ds(E + base, C)], iv)
            cps = [
                pltpu.async_copy(px.at[jv], xj, sem_j),
                pltpu.async_copy(py.at[jv], yj, sem_j),
                pltpu.async_copy(pz.at[jv], zj, sem_j),
                pltpu.async_copy(px.at[iv], xi, sem_i),
                pltpu.async_copy(py.at[iv], yi, sem_i),
                pltpu.async_copy(pz.at[iv], zi, sem_i),
            ]
            for cp in cps:
                cp.wait()

            def grp(g, carry2):
                sl = pl.ds(g * _L, _L)
                dx = xj[sl] - xi[sl]
                dy = yj[sl] - yi[sl]
                dz = zj[sl] - zi[sl]
                s = dx * dx + dy * dy + dz * dz
                r = _rsqrt(s)
                offv = (g >> 3) * (4 * _B) + (g & 7) * _L + lane
                plsc.store_scatter(ov, [offv], s * r)
                plsc.store_scatter(ov, [offv + _B], dx * r)
                plsc.store_scatter(ov, [offv + 2 * _B], dy * r)
                plsc.store_scatter(ov, [offv + 3 * _B], dz * r)
                return carry2

            lax.fori_loop(0, C // _L, grp, 0)
            pltpu.sync_copy(ov, out.at[pl.ds(4 * base, 4 * C)])

        return carry

    lax.fori_loop(0, n_steps, chunk, 0)


def kernel(pos, edge_index):
    N = pos.shape[0]
    E = edge_index.shape[1]
    C = 2560
    assert E % C == 0 and C % _B == 0 and C % _L == 0
    assert E // C >= 2 * _W
    post = pos.T

    mesh = plsc.VectorSubcoreMesh(core_axis_name="c", subcore_axis_name="s")
    k = pl.kernel(
        functools.partial(_edge_body, E, C),
        out_type=jax.ShapeDtypeStruct((4 * E,), jnp.float32),
        mesh=mesh,
        compiler_params=pltpu.CompilerParams(
            needs_layout_passes=False, use_tc_tiling_on_sc=False),
        scratch_types=[
            pltpu.VMEM((C,), jnp.int32),
            pltpu.VMEM((C,), jnp.int32),
            pltpu.VMEM((C,), jnp.float32),
            pltpu.VMEM((C,), jnp.float32),
            pltpu.VMEM((C,), jnp.float32),
            pltpu.VMEM((C,), jnp.float32),
            pltpu.VMEM((C,), jnp.float32),
            pltpu.VMEM((C,), jnp.float32),
            pltpu.VMEM((4 * C,), jnp.float32),
            pltpu.SemaphoreType.DMA,
            pltpu.SemaphoreType.DMA,
        ],
    )
    out_flat = k(edge_index.reshape(2 * E), post[0], post[1], post[2])
    return out_flat.reshape(E // _B, 4, _B).swapaxes(1, 2).reshape(E, 4)


# speedup vs baseline: 29.7565x; 1.0724x over previous
import functools

import jax
import jax.numpy as jnp
from jax import lax
from jax.experimental import pallas as pl
from jax.experimental.pallas import tpu as pltpu
from jax.experimental.pallas import tpu_sc as plsc

_NC = 2
_NS = 16
_W = _NC * _NS
_L = 16
_B = 128


def _rsqrt(s):
    bits = lax.bitcast_convert_type(s, jnp.int32)
    bits = jnp.int32(0x5F3759DF) - (bits >> 1)
    y = lax.bitcast_convert_type(bits, jnp.float32)
    for _ in range(3):
        y = y * (1.5 - 0.5 * s * y * y)
    return y


def _edge_body(E, C, eidx, px, py, pz, out,
               jv, iv, gx, ov, sem_idx, sem_g, sem_out):
    n_chunks = E // C
    n_steps = (n_chunks + _W - 1) // _W
    wid = lax.axis_index("s") * _NC + lax.axis_index("c")
    lane = lax.iota(jnp.int32, _L)

    def gchunk(t):
        return wid + t * _W

    def valid(t):
        return gchunk(t) < n_chunks

    def idx_copy(t, b):
        base = gchunk(t) * C
        pltpu.async_copy(eidx.at[pl.ds(base, C)], jv[b], sem_idx[b])
        pltpu.async_copy(eidx.at[pl.ds(E + base, C)], iv[b], sem_idx[b])

    def idx_wait(b):
        pltpu.make_async_copy(eidx.at[pl.ds(0, C)], jv[b], sem_idx[b]).wait()
        pltpu.make_async_copy(eidx.at[pl.ds(0, C)], iv[b], sem_idx[b]).wait()

    def gather(b):
        pltpu.async_copy(px.at[jv[b]], gx[b].at[0], sem_g[b])
        pltpu.async_copy(py.at[jv[b]], gx[b].at[1], sem_g[b])
        pltpu.async_copy(pz.at[jv[b]], gx[b].at[2], sem_g[b])
        pltpu.async_copy(px.at[iv[b]], gx[b].at[3], sem_g[b])
        pltpu.async_copy(py.at[iv[b]], gx[b].at[4], sem_g[b])
        pltpu.async_copy(pz.at[iv[b]], gx[b].at[5], sem_g[b])

    def gather_wait(b):
        for p in range(6):
            pltpu.make_async_copy(px.at[jv[b]], gx[b].at[p], sem_g[b]).wait()

    def out_copy(t, b):
        pltpu.async_copy(ov[b], out.at[pl.ds(4 * gchunk(t) * C, 4 * C)],
                         sem_out[b])

    def out_wait(b):
        pltpu.make_async_copy(ov[b], out.at[pl.ds(0, 4 * C)],
                              sem_out[b]).wait()

    def compute(b):
        def grp(g, carry2):
            sl = pl.ds(g * _L, _L)
            dx = gx[b][0, sl] - gx[b][3, sl]
            dy = gx[b][1, sl] - gx[b][4, sl]
            dz = gx[b][2, sl] - gx[b][5, sl]
            s = dx * dx + dy * dy + dz * dz
            r = _rsqrt(s)
            offv = (g >> 3) * (4 * _B) + (g & 7) * _L + lane
            plsc.store_scatter(ov[b], [offv], s * r)
            plsc.store_scatter(ov[b], [offv + _B], dx * r)
            plsc.store_scatter(ov[b], [offv + 2 * _B], dy * r)
            plsc.store_scatter(ov[b], [offv + 3 * _B], dz * r)
            return carry2

        lax.fori_loop(0, C // _L, grp, 0)

    def step(t, b):
        nb = 1 - b

        @pl.when(valid(t + 1))
        def _():
            idx_wait(nb)
            gather(nb)

        @pl.when(valid(t))
        def _():
            gather_wait(b)

            @pl.when(t >= 2)
            def _():
                out_wait(b)

            compute(b)
            out_copy(t, b)

        @pl.when(valid(t + 2))
        def _():
            idx_copy(t + 2, b)

    idx_copy(0, 0)
    idx_copy(1, 1)
    idx_wait(0)
    gather(0)

    def pair(p, carry):
        step(2 * p, 0)
        step(2 * p + 1, 1)
        return carry

    lax.fori_loop(0, n_steps // 2, pair, 0)
    if n_steps % 2:
        step(n_steps - 1, 0)
    out_wait(1 if n_steps % 2 else 0)
    out_wait(0 if n_steps % 2 else 1)


def kernel(pos, edge_index):
    N = pos.shape[0]
    E = edge_index.shape[1]
    C = 2560
    assert E % C == 0 and C % _B == 0 and C % _L == 0
    assert E // C >= 2 * _W
    post = pos.T

    mesh = plsc.VectorSubcoreMesh(core_axis_name="c", subcore_axis_name="s")
    k = pl.kernel(
        functools.partial(_edge_body, E, C),
        out_type=jax.ShapeDtypeStruct((4 * E,), jnp.float32),
        mesh=mesh,
        compiler_params=pltpu.CompilerParams(
            needs_layout_passes=False, use_tc_tiling_on_sc=False),
        scratch_types=[
            [pltpu.VMEM((C,), jnp.int32)] * 2,
            [pltpu.VMEM((C,), jnp.int32)] * 2,
            [pltpu.VMEM((6, C), jnp.float32)] * 2,
            [pltpu.VMEM((4 * C,), jnp.float32)] * 2,
            [pltpu.SemaphoreType.DMA] * 2,
            [pltpu.SemaphoreType.DMA] * 2,
            [pltpu.SemaphoreType.DMA] * 2,
        ],
    )
    out_flat = k(edge_index.reshape(2 * E), post[0], post[1], post[2])
    return out_flat.reshape(E // _B, 4, _B).swapaxes(1, 2).reshape(E, 4)


# speedup vs baseline: 98.5474x; 3.3118x over previous
import functools

import jax
import jax.numpy as jnp
from jax import lax
from jax.experimental import pallas as pl
from jax.experimental.pallas import tpu as pltpu
from jax.experimental.pallas import tpu_sc as plsc

_NC = 2
_NS = 16
_W = _NC * _NS
_L = 16
_B = 128


def _rsqrt(s):
    bits = lax.bitcast_convert_type(s, jnp.int32)
    bits = jnp.int32(0x5F3759DF) - (bits >> 1)
    y = lax.bitcast_convert_type(bits, jnp.float32)
    for _ in range(3):
        y = y * (1.5 - 0.5 * s * y * y)
    return y


def _edge_body(E, C, eidx, px, py, pz, out,
               jv, iv, gx, ov, sx, sy, sz, sem_idx, sem_g, sem_out):
    n_chunks = E // C
    n_steps = (n_chunks + _W - 1) // _W
    wid = lax.axis_index("s") * _NC + lax.axis_index("c")
    lane = lax.iota(jnp.int32, _L)

    @pl.when(lax.axis_index("s") == 0)
    def _():
        pltpu.sync_copy(px, sx)
        pltpu.sync_copy(py, sy)
        pltpu.sync_copy(pz, sz)

    plsc.subcore_barrier()

    def gchunk(t):
        return wid + t * _W

    def valid(t):
        return gchunk(t) < n_chunks

    def idx_copy(t, b):
        base = gchunk(t) * C
        pltpu.async_copy(eidx.at[pl.ds(base, C)], jv[b], sem_idx[b])
        pltpu.async_copy(eidx.at[pl.ds(E + base, C)], iv[b], sem_idx[b])

    def idx_wait(b):
        pltpu.make_async_copy(eidx.at[pl.ds(0, C)], jv[b], sem_idx[b]).wait()
        pltpu.make_async_copy(eidx.at[pl.ds(0, C)], iv[b], sem_idx[b]).wait()

    def gather(b):
        pltpu.async_copy(sx.at[jv[b]], gx[b].at[0], sem_g[b])
        pltpu.async_copy(sy.at[jv[b]], gx[b].at[1], sem_g[b])
        pltpu.async_copy(sz.at[jv[b]], gx[b].at[2], sem_g[b])
        pltpu.async_copy(sx.at[iv[b]], gx[b].at[3], sem_g[b])
        pltpu.async_copy(sy.at[iv[b]], gx[b].at[4], sem_g[b])
        pltpu.async_copy(sz.at[iv[b]], gx[b].at[5], sem_g[b])

    def gather_wait(b):
        for p in range(6):
            pltpu.make_async_copy(sx.at[jv[b]], gx[b].at[p], sem_g[b]).wait()

    def out_copy(t, b):
        pltpu.async_copy(ov[b], out.at[pl.ds(4 * gchunk(t) * C, 4 * C)],
                         sem_out[b])

    def out_wait(b):
        pltpu.make_async_copy(ov[b], out.at[pl.ds(0, 4 * C)],
                              sem_out[b]).wait()

    def compute(b):
        def grp(g, carry2):
            sl = pl.ds(g * _L, _L)
            dx = gx[b][0, sl] - gx[b][3, sl]
            dy = gx[b][1, sl] - gx[b][4, sl]
            dz = gx[b][2, sl] - gx[b][5, sl]
            s = dx * dx + dy * dy + dz * dz
            r = _rsqrt(s)
            offv = (g >> 3) * (4 * _B) + (g & 7) * _L + lane
            plsc.store_scatter(ov[b], [offv], s * r)
            plsc.store_scatter(ov[b], [offv + _B], dx * r)
            plsc.store_scatter(ov[b], [offv + 2 * _B], dy * r)
            plsc.store_scatter(ov[b], [offv + 3 * _B], dz * r)
            return carry2

        lax.fori_loop(0, C // _L, grp, 0)

    def step(t, b):
        nb = 1 - b

        @pl.when(valid(t + 1))
        def _():
            idx_wait(nb)
            gather(nb)

        @pl.when(valid(t))
        def _():
            gather_wait(b)

            @pl.when(t >= 2)
            def _():
                out_wait(b)

            compute(b)
            out_copy(t, b)

        @pl.when(valid(t + 2))
        def _():
            idx_copy(t + 2, b)

    idx_copy(0, 0)
    idx_copy(1, 1)
    idx_wait(0)
    gather(0)

    def pair(p, carry):
        step(2 * p, 0)
        step(2 * p + 1, 1)
        return carry

    lax.fori_loop(0, n_steps // 2, pair, 0)
    if n_steps % 2:
        step(n_steps - 1, 0)
    out_wait(1 if n_steps % 2 else 0)
    out_wait(0 if n_steps % 2 else 1)


def kernel(pos, edge_index):
    N = pos.shape[0]
    E = edge_index.shape[1]
    C = 2560
    assert E % C == 0 and C % _B == 0 and C % _L == 0
    assert E // C >= 2 * _W
    post = pos.T

    mesh = plsc.VectorSubcoreMesh(core_axis_name="c", subcore_axis_name="s")
    k = pl.kernel(
        functools.partial(_edge_body, E, C),
        name="edge_props",
        out_type=jax.ShapeDtypeStruct((4 * E,), jnp.float32),
        mesh=mesh,
        compiler_params=pltpu.CompilerParams(
            needs_layout_passes=False, use_tc_tiling_on_sc=False),
        scratch_types=[
            [pltpu.VMEM((C,), jnp.int32)] * 2,
            [pltpu.VMEM((C,), jnp.int32)] * 2,
            [pltpu.VMEM((6, C), jnp.float32)] * 2,
            [pltpu.VMEM((4 * C,), jnp.float32)] * 2,
            pltpu.VMEM_SHARED((N,), jnp.float32),
            pltpu.VMEM_SHARED((N,), jnp.float32),
            pltpu.VMEM_SHARED((N,), jnp.float32),
            [pltpu.SemaphoreType.DMA] * 2,
            [pltpu.SemaphoreType.DMA] * 2,
            [pltpu.SemaphoreType.DMA] * 2,
        ],
    )
    out_flat = k(edge_index.reshape(2 * E), post[0], post[1], post[2])
    return out_flat.reshape(E // _B, 4, _B).swapaxes(1, 2).reshape(E, 4)


# speedup vs baseline: 114.7392x; 1.1643x over previous
import functools

import jax
import jax.numpy as jnp
from jax import lax
from jax.experimental import pallas as pl
from jax.experimental.pallas import tpu as pltpu
from jax.experimental.pallas import tpu_sc as plsc

_NC = 2
_NS = 16
_W = _NC * _NS
_L = 16
_B = 128


def _rsqrt(s):
    bits = lax.bitcast_convert_type(s, jnp.int32)
    bits = jnp.int32(0x5F3759DF) - (bits >> 1)
    y = lax.bitcast_convert_type(bits, jnp.float32)
    for _ in range(2):
        y = y * (1.5 - 0.5 * s * y * y)
    return y


def _edge_body(E, C, eidx, px, py, pz, out,
               jv, iv, gx, ov, sx, sy, sz, sem_idx, sem_g, sem_out):
    n_chunks = E // C
    n_steps = (n_chunks + _W - 1) // _W
    wid = lax.axis_index("s") * _NC + lax.axis_index("c")
    lane = lax.iota(jnp.int32, _L)

    @pl.when(lax.axis_index("s") == 0)
    def _():
        pltpu.sync_copy(px, sx)
        pltpu.sync_copy(py, sy)
        pltpu.sync_copy(pz, sz)

    plsc.subcore_barrier()

    def gchunk(t):
        return wid + t * _W

    def valid(t):
        return gchunk(t) < n_chunks

    def idx_copy(t, b):
        base = gchunk(t) * C
        pltpu.async_copy(eidx.at[pl.ds(base, C)], jv[b], sem_idx[b])
        pltpu.async_copy(eidx.at[pl.ds(E + base, C)], iv[b], sem_idx[b])

    def idx_wait(b):
        pltpu.make_async_copy(eidx.at[pl.ds(0, C)], jv[b], sem_idx[b]).wait()
        pltpu.make_async_copy(eidx.at[pl.ds(0, C)], iv[b], sem_idx[b]).wait()

    def gather(b):
        pltpu.async_copy(sx.at[jv[b]], gx[b].at[0], sem_g[b])
        pltpu.async_copy(sy.at[jv[b]], gx[b].at[1], sem_g[b])
        pltpu.async_copy(sz.at[jv[b]], gx[b].at[2], sem_g[b])
        pltpu.async_copy(sx.at[iv[b]], gx[b].at[3], sem_g[b])
        pltpu.async_copy(sy.at[iv[b]], gx[b].at[4], sem_g[b])
        pltpu.async_copy(sz.at[iv[b]], gx[b].at[5], sem_g[b])

    def gather_wait(b):
        for p in range(6):
            pltpu.make_async_copy(sx.at[jv[b]], gx[b].at[p], sem_g[b]).wait()

    def out_copy(t, b):
        pltpu.async_copy(ov[b], out.at[pl.ds(4 * gchunk(t) * C, 4 * C)],
                         sem_out[b])

    def out_wait(b):
        pltpu.make_async_copy(ov[b], out.at[pl.ds(0, 4 * C)],
                              sem_out[b]).wait()

    def compute(b):
        def one(g):
            sl = pl.ds(g * _L, _L)
            dx = gx[b][0, sl] - gx[b][3, sl]
            dy = gx[b][1, sl] - gx[b][4, sl]
            dz = gx[b][2, sl] - gx[b][5, sl]
            s = dx * dx + dy * dy + dz * dz
            r = _rsqrt(s)
            offv = (g >> 3) * (4 * _B) + (g & 7) * _L + lane
            plsc.store_scatter(ov[b], [offv], s * r)
            plsc.store_scatter(ov[b], [offv + _B], dx * r)
            plsc.store_scatter(ov[b], [offv + 2 * _B], dy * r)
            plsc.store_scatter(ov[b], [offv + 3 * _B], dz * r)

        def grp(q, carry2):
            one(2 * q)
            one(2 * q + 1)
            return carry2

        lax.fori_loop(0, C // (2 * _L), grp, 0)

    def step(t, b):
        nb = 1 - b

        @pl.when(valid(t + 1))
        def _():
            idx_wait(nb)
            gather(nb)

        @pl.when(valid(t))
        def _():
            gather_wait(b)

            @pl.when(t >= 2)
            def _():
                out_wait(b)

            compute(b)
            out_copy(t, b)

        @pl.when(valid(t + 2))
        def _():
            idx_copy(t + 2, b)

    idx_copy(0, 0)
    idx_copy(1, 1)
    idx_wait(0)
    gather(0)

    def pair(p, carry):
        step(2 * p, 0)
        step(2 * p + 1, 1)
        return carry

    lax.fori_loop(0, n_steps // 2, pair, 0)
    if n_steps % 2:
        step(n_steps - 1, 0)
    out_wait(1 if n_steps % 2 else 0)
    out_wait(0 if n_steps % 2 else 1)


def kernel(pos, edge_index):
    N = pos.shape[0]
    E = edge_index.shape[1]
    C = 3200
    assert E % C == 0 and C % _B == 0 and C % _L == 0
    assert E // C >= 2 * _W
    post = pos.T

    mesh = plsc.VectorSubcoreMesh(core_axis_name="c", subcore_axis_name="s")
    k = pl.kernel(
        functools.partial(_edge_body, E, C),
        name="edge_props",
        out_type=jax.ShapeDtypeStruct((4 * E,), jnp.float32),
        mesh=mesh,
        compiler_params=pltpu.CompilerParams(
            needs_layout_passes=False, use_tc_tiling_on_sc=False),
        scratch_types=[
            [pltpu.VMEM((C,), jnp.int32)] * 2,
            [pltpu.VMEM((C,), jnp.int32)] * 2,
            [pltpu.VMEM((6, C), jnp.float32)] * 2,
            [pltpu.VMEM((4 * C,), jnp.float32)] * 2,
            pltpu.VMEM_SHARED((N,), jnp.float32),
            pltpu.VMEM_SHARED((N,), jnp.float32),
            pltpu.VMEM_SHARED((N,), jnp.float32),
            [pltpu.SemaphoreType.DMA] * 2,
            [pltpu.SemaphoreType.DMA] * 2,
            [pltpu.SemaphoreType.DMA] * 2,
        ],
    )
    out_flat = k(edge_index.reshape(2 * E), post[0], post[1], post[2])
    return out_flat.reshape(E // _B, 4, _B).swapaxes(1, 2).reshape(E, 4)
